# baseline (speedup 1.0000x reference)
import functools

import jax
import jax.numpy as jnp
import numpy as np
from jax import lax
from jax.experimental import pallas as pl
from jax.experimental.pallas import tpu as pltpu
from jax.experimental.pallas import tpu_sc as plsc

NUM_CLASSES = 1000
B0 = 4096
B1 = 26
L = 16
NC = 2
NW = 32
D0_PER_W = B0 // NW
N_CH = D0_PER_W
XE_W = 2 * L
ROW_W = B1 * NUM_CLASSES

_CONSTS = np.concatenate([np.zeros(L, np.int32), np.ones(L, np.int32)])


def _sc_onehot(pe_hbm, c_hbm, z_hbm, o_hbm, buf_a, buf_b, pe_l, c_l,
               sem_a, sem_b):
    wid = lax.axis_index("s") * NC + lax.axis_index("c")
    base_d = wid * D0_PER_W

    pltpu.sync_copy(pe_hbm.at[pl.ds(base_d * XE_W, D0_PER_W * XE_W)], pe_l)
    pltpu.sync_copy(c_hbm, c_l)
    pltpu.sync_copy(z_hbm, buf_a)
    pltpu.sync_copy(z_hbm, buf_b)

    def scatter_val(buf, c, val_slot):
        vals = c_l[pl.ds(val_slot * L, L)]
        for v in range(2):
            pos = pe_l[pl.ds(c * XE_W + v * L, L)]
            plsc.store_scatter(buf, [pos], vals)

    def chunk(c, buf, sem):
        d0 = base_d + c

        @pl.when(c >= 2)
        def _wait_and_clear():
            pltpu.make_async_copy(
                buf, o_hbm.at[pl.ds((d0 - 2) * ROW_W, ROW_W)], sem
            ).wait()
            scatter_val(buf, c - 2, 0)

        scatter_val(buf, c, 1)
        pltpu.async_copy(buf, o_hbm.at[pl.ds(d0 * ROW_W, ROW_W)], sem)

    def outer(i, carry):
        chunk(2 * i, buf_a, sem_a)
        chunk(2 * i + 1, buf_b, sem_b)
        return carry

    lax.fori_loop(0, N_CH // 2, outer, 0)
    pltpu.make_async_copy(
        buf_a, o_hbm.at[pl.ds((base_d + N_CH - 2) * ROW_W, ROW_W)], sem_a
    ).wait()
    pltpu.make_async_copy(
        buf_b, o_hbm.at[pl.ds((base_d + N_CH - 1) * ROW_W, ROW_W)], sem_b
    ).wait()


@jax.jit
def _onehot_sc(x):
    pos = x + jnp.arange(B1, dtype=jnp.int32)[None, :] * NUM_CLASSES
    pe = jnp.concatenate([pos, pos[:, : XE_W - B1]], axis=1).reshape(
        B0 * XE_W
    )
    consts = jnp.asarray(_CONSTS)
    z = jnp.zeros((ROW_W,), jnp.int32)
    run = functools.partial(
        pl.kernel,
        mesh=plsc.VectorSubcoreMesh(core_axis_name="c", subcore_axis_name="s"),
        compiler_params=pltpu.CompilerParams(
            use_tc_tiling_on_sc=False, needs_layout_passes=False
        ),
        out_type=jax.ShapeDtypeStruct((B0 * B1 * NUM_CLASSES,), jnp.int32),
        scratch_types=[
            pltpu.VMEM((ROW_W,), jnp.int32),
            pltpu.VMEM((ROW_W,), jnp.int32),
            pltpu.VMEM((D0_PER_W * XE_W,), jnp.int32),
            pltpu.VMEM((len(_CONSTS),), jnp.int32),
            pltpu.SemaphoreType.DMA,
            pltpu.SemaphoreType.DMA,
        ],
    )(_sc_onehot)
    out = run(pe, consts, z)
    return out.reshape(B0, B1, NUM_CLASSES)


def kernel(x):
    return _onehot_sc(x)
